# baseline (speedup 1.0000x reference)
import jax
import jax.numpy as jnp
from jax.experimental import pallas as pl
from jax.experimental.pallas import tpu as pltpu

_D = 512
_CH = 128
_BM = 4096

_HI = jax.lax.Precision.HIGHEST


def _dot(a, b, prec=_HI):
    return jnp.dot(a, b, preferred_element_type=jnp.float32, precision=prec)


def _dot_ta(a, b, prec=_HI):
    return jax.lax.dot_general(a, b, (((0,), (0,)), ((), ())),
                               preferred_element_type=jnp.float32,
                               precision=prec)


def _dot_tb(a, b, prec=_HI):
    return jax.lax.dot_general(a, b, (((1,), (1,)), ((), ())),
                               preferred_element_type=jnp.float32,
                               precision=prec)


def _chunk_t(G, row, col):
    T = jnp.where(row == col, 2.0, 0.0) + jnp.where(
        (row % 2 == 0) & (col == row + 1), -4.0 * G, 0.0)
    m = 2
    while m < _CH:
        mask = ((row // (2 * m) == col // (2 * m))
                & (row % (2 * m) < m) & (col % (2 * m) >= m))
        A = _dot(_dot(T, G), T)
        T = T - jnp.where(mask, A, 0.0)
        m *= 2
    return T


def _build_b(V):
    norm = jnp.sqrt(jnp.sum(V * V, axis=1, keepdims=True)) + 1e-6
    Vn = V / norm

    row = jax.lax.broadcasted_iota(jnp.int32, (_CH, _CH), 0)
    col = jax.lax.broadcasted_iota(jnp.int32, (_CH, _CH), 1)

    Ws, Ts = [], []
    for k in range(_D // _CH):
        Wk = Vn[k * _CH:(k + 1) * _CH, :]
        Gk = _dot_tb(Wk, Wk)
        Ws.append(Wk)
        Ts.append(_chunk_t(Gk, row, col))

    Z = _dot_ta(Ts[0], Ws[0])
    rowd = jax.lax.broadcasted_iota(jnp.int32, (_D, _D), 0)
    cold = jax.lax.broadcasted_iota(jnp.int32, (_D, _D), 1)
    P = jnp.where(rowd == cold, 1.0, 0.0) - _dot_ta(Ws[0], Z)
    for k in range(1, _D // _CH):
        WkP = _dot(Ws[k], P)
        Z = _dot_ta(Ts[k], WkP)
        P = P - _dot_ta(Ws[k], Z)
    return P


def _fused_kernel(v_ref, x_ref, o_ref, b_scr):
    i = pl.program_id(0)
    half = pl.num_programs(0) // 2

    @pl.when((i == 0) | (i == half))
    def _():
        b_scr[...] = _build_b(v_ref[0])

    o_ref[...] = jnp.dot(x_ref[...], b_scr[...],
                         preferred_element_type=jnp.float32)


def kernel(x, hd_vecs):
    n, d = x.shape
    assert d == _D

    out = pl.pallas_call(
        _fused_kernel,
        out_shape=jax.ShapeDtypeStruct((n, d), x.dtype),
        grid=(n // _BM,),
        in_specs=[
            pl.BlockSpec((1, _D, _D), lambda i: (0, 0, 0)),
            pl.BlockSpec((_BM, d), lambda i: (i, 0)),
        ],
        out_specs=pl.BlockSpec((_BM, d), lambda i: (i, 0)),
        scratch_shapes=[pltpu.VMEM((_D, _D), jnp.float32)],
        compiler_params=pltpu.CompilerParams(
            dimension_semantics=("parallel",)),
    )(hd_vecs, x)
    return out


# speedup vs baseline: 1.0921x; 1.0921x over previous
import jax
import jax.numpy as jnp
from jax.experimental import pallas as pl
from jax.experimental.pallas import tpu as pltpu

_D = 512
_CH = 128
_BM = 4096

_HI = jax.lax.Precision.HIGHEST


def _dot(a, b, prec=_HI):
    return jnp.dot(a, b, preferred_element_type=jnp.float32, precision=prec)


def _dot_ta(a, b, prec=_HI):
    return jax.lax.dot_general(a, b, (((0,), (0,)), ((), ())),
                               preferred_element_type=jnp.float32,
                               precision=prec)


def _dot_tb(a, b, prec=_HI):
    return jax.lax.dot_general(a, b, (((1,), (1,)), ((), ())),
                               preferred_element_type=jnp.float32,
                               precision=prec)


def _chunk_t(G, row, col):
    T = jnp.where(row == col, 2.0, 0.0) + jnp.where(
        (row % 2 == 0) & (col == row + 1), -4.0 * G, 0.0)
    m = 2
    while m < _CH:
        mask = ((row // (2 * m) == col // (2 * m))
                & (row % (2 * m) < m) & (col % (2 * m) >= m))
        A = _dot(_dot(T, G), T)
        T = T - jnp.where(mask, A, 0.0)
        m *= 2
    return T


def _merge(T1, T2, G12, m):
    C = -_dot(_dot(T1, G12), T2)
    top = jnp.concatenate([T1, C], axis=1)
    bot = jnp.concatenate([jnp.zeros((m, m), jnp.float32), T2], axis=1)
    return jnp.concatenate([top, bot], axis=0)


def _build_b_kernel(v_ref, b_ref):
    V = v_ref[0]
    norm = jnp.sqrt(jnp.sum(V * V, axis=1, keepdims=True)) + 1e-6
    Vn = V / norm

    G = _dot_tb(Vn, Vn)

    row = jax.lax.broadcasted_iota(jnp.int32, (_CH, _CH), 0)
    col = jax.lax.broadcasted_iota(jnp.int32, (_CH, _CH), 1)

    Ts = [_chunk_t(G[k * _CH:(k + 1) * _CH, k * _CH:(k + 1) * _CH], row, col)
          for k in range(_D // _CH)]

    T12 = _merge(Ts[0], Ts[1], G[0:128, 128:256], _CH)
    T34 = _merge(Ts[2], Ts[3], G[256:384, 384:512], _CH)
    T = _merge(T12, T34, G[0:256, 256:512], 2 * _CH)

    C = _dot(T, Vn)
    CtVn = _dot_ta(C, Vn)
    rowd = jax.lax.broadcasted_iota(jnp.int32, (_D, _D), 0)
    cold = jax.lax.broadcasted_iota(jnp.int32, (_D, _D), 1)
    b_ref[...] = jnp.where(rowd == cold, 1.0, 0.0) - CtVn


def _apply_kernel(x_ref, b_ref, o_ref):
    o_ref[...] = jnp.dot(x_ref[...], b_ref[...],
                         preferred_element_type=jnp.float32)


def kernel(x, hd_vecs):
    n, d = x.shape
    assert d == _D

    B = pl.pallas_call(
        _build_b_kernel,
        out_shape=jax.ShapeDtypeStruct((_D, _D), jnp.float32),
        in_specs=[pl.BlockSpec((1, _D, _D), lambda: (0, 0, 0))],
        out_specs=pl.BlockSpec((_D, _D), lambda: (0, 0)),
    )(hd_vecs)

    out = pl.pallas_call(
        _apply_kernel,
        out_shape=jax.ShapeDtypeStruct((n, d), x.dtype),
        grid=(n // _BM,),
        in_specs=[
            pl.BlockSpec((_BM, d), lambda i: (i, 0)),
            pl.BlockSpec((_D, _D), lambda i: (0, 0)),
        ],
        out_specs=pl.BlockSpec((_BM, d), lambda i: (i, 0)),
        compiler_params=pltpu.CompilerParams(
            dimension_semantics=("parallel",)),
    )(x, B)
    return out


# speedup vs baseline: 1.1257x; 1.0308x over previous
import jax
import jax.numpy as jnp
from jax.experimental import pallas as pl
from jax.experimental.pallas import tpu as pltpu

_D = 512
_CH = 128
_BM = 4096

_NN = (((1,), (0,)), ((), ()))
_TA = (((0,), (0,)), ((), ()))
_TB = (((1,), (1,)), ((), ()))


def _sp(a):
    hi = a.astype(jnp.bfloat16)
    lo = (a - hi.astype(jnp.float32)).astype(jnp.bfloat16)
    return hi, lo


def _bdot(a, b, dims=_NN):
    ah, al = a
    bh, bl = b
    d = lambda u, v: jax.lax.dot_general(u, v, dims,
                                         preferred_element_type=jnp.float32)
    return d(ah, bh) + d(ah, bl) + d(al, bh)


def _chunk_t(G, Gsp, row, col):
    T = jnp.where(row == col, 2.0, 0.0) + jnp.where(
        (row % 2 == 0) & (col == row + 1), -4.0 * G, 0.0)
    m = 2
    while m < _CH:
        mask = ((row // (2 * m) == col // (2 * m))
                & (row % (2 * m) < m) & (col % (2 * m) >= m))
        Tsp = _sp(T)
        A = _bdot(_sp(_bdot(Tsp, Gsp)), Tsp)
        T = T - jnp.where(mask, A, 0.0)
        m *= 2
    return T


def _merge(T1, T2, G12sp, m):
    C = -_bdot(_sp(_bdot(_sp(T1), G12sp)), _sp(T2))
    top = jnp.concatenate([T1, C], axis=1)
    bot = jnp.concatenate([jnp.zeros((m, m), jnp.float32), T2], axis=1)
    return jnp.concatenate([top, bot], axis=0)


def _build_b_kernel(v_ref, b_ref):
    V = v_ref[0]
    norm = jnp.sqrt(jnp.sum(V * V, axis=1, keepdims=True)) + 1e-6
    Vn = V / norm
    Vnsp = _sp(Vn)

    G = _bdot(Vnsp, Vnsp, _TB)
    sl = lambda r0, r1, c0, c1: (G[r0:r1, c0:c1].astype(jnp.bfloat16),
                                 (G[r0:r1, c0:c1]
                                  - G[r0:r1, c0:c1].astype(jnp.bfloat16)
                                  .astype(jnp.float32)).astype(jnp.bfloat16))

    row = jax.lax.broadcasted_iota(jnp.int32, (_CH, _CH), 0)
    col = jax.lax.broadcasted_iota(jnp.int32, (_CH, _CH), 1)

    Ts = [_chunk_t(G[k * _CH:(k + 1) * _CH, k * _CH:(k + 1) * _CH],
                   sl(k * _CH, (k + 1) * _CH, k * _CH, (k + 1) * _CH),
                   row, col)
          for k in range(_D // _CH)]

    T12 = _merge(Ts[0], Ts[1], sl(0, 128, 128, 256), _CH)
    T34 = _merge(Ts[2], Ts[3], sl(256, 384, 384, 512), _CH)
    T = _merge(T12, T34, sl(0, 256, 256, 512), 2 * _CH)

    C = _bdot(_sp(T), Vnsp)
    CtVn = _bdot(_sp(C), Vnsp, _TA)
    rowd = jax.lax.broadcasted_iota(jnp.int32, (_D, _D), 0)
    cold = jax.lax.broadcasted_iota(jnp.int32, (_D, _D), 1)
    b_ref[...] = jnp.where(rowd == cold, 1.0, 0.0) - CtVn


def _apply_kernel(x_ref, b_ref, o_ref):
    o_ref[...] = jnp.dot(x_ref[...], b_ref[...],
                         preferred_element_type=jnp.float32)


def kernel(x, hd_vecs):
    n, d = x.shape
    assert d == _D

    B = pl.pallas_call(
        _build_b_kernel,
        out_shape=jax.ShapeDtypeStruct((_D, _D), jnp.float32),
        in_specs=[pl.BlockSpec((1, _D, _D), lambda: (0, 0, 0))],
        out_specs=pl.BlockSpec((_D, _D), lambda: (0, 0)),
    )(hd_vecs)

    out = pl.pallas_call(
        _apply_kernel,
        out_shape=jax.ShapeDtypeStruct((n, d), x.dtype),
        grid=(n // _BM,),
        in_specs=[
            pl.BlockSpec((_BM, d), lambda i: (i, 0)),
            pl.BlockSpec((_D, _D), lambda i: (0, 0)),
        ],
        out_specs=pl.BlockSpec((_BM, d), lambda i: (i, 0)),
        compiler_params=pltpu.CompilerParams(
            dimension_semantics=("parallel",)),
    )(x, B)
    return out
